# Initial kernel scaffold; baseline (speedup 1.0000x reference)
import jax
import jax.numpy as jnp
from jax.experimental import pallas as pl


def kernel(x, edge_index, edge_weights, W, b):
    raise NotImplementedError("write your pallas kernel here")


# speedup vs baseline: 7.8978x; 7.8978x over previous
import functools

import jax
import jax.numpy as jnp
from jax import lax
from jax.experimental import pallas as pl
from jax.experimental.pallas import tpu as pltpu
from jax.experimental.pallas import tpu_sc as plsc

N = 10000
E = 320000
D = 128

NC = 2
NS = 16
NW = NC * NS

CH = 128
E_TILE = 10240
NCHUNK = E_TILE // CH
E_HALF = E_TILE * NS
E_PAD = E_HALF * NC
N_PAD = 10240
ROWS_T = N_PAD // NS
DUMMY = N

_mesh = plsc.VectorSubcoreMesh(core_axis_name="c", subcore_axis_name="s")


@functools.partial(
    pl.kernel,
    out_type=(jax.ShapeDtypeStruct((N_PAD,), jnp.float32),
              jax.ShapeDtypeStruct((N_PAD,), jnp.float32)),
    mesh=_mesh,
    scratch_types=[
        pltpu.VMEM((CH,), jnp.int32),
        pltpu.VMEM((CH,), jnp.float32),
        pltpu.VMEM_SHARED((N_PAD,), jnp.float32),
    ],
)
def _deg_kernel(dst_hbm, zeros1_hbm, d0_hbm, d1_hbm, idx_v, ones_v, dacc):
    c = lax.axis_index("c")
    s = lax.axis_index("s")
    sl = pl.ds(s * ROWS_T, ROWS_T)
    pltpu.sync_copy(zeros1_hbm.at[sl], dacc.at[sl])

    def fill(i, _):
        ones_v[pl.ds(i * 16, 16)] = jnp.ones((16,), jnp.float32)
        return 0
    lax.fori_loop(0, CH // 16, fill, 0)
    plsc.subcore_barrier()

    base = c * E_HALF + s * E_TILE

    def body(i, _):
        pltpu.sync_copy(dst_hbm.at[pl.ds(base + i * CH, CH)], idx_v)
        pltpu.sync_copy(ones_v, dacc.at[idx_v], add=True)
        return 0
    lax.fori_loop(0, NCHUNK, body, 0)
    plsc.subcore_barrier()

    @pl.when(c == 0)
    def _():
        pltpu.sync_copy(dacc.at[sl], d0_hbm.at[sl])

    @pl.when(c == 1)
    def _():
        pltpu.sync_copy(dacc.at[sl], d1_hbm.at[sl])


@functools.partial(
    pl.kernel,
    out_type=(jax.ShapeDtypeStruct((N_PAD, D), jnp.float32),
              jax.ShapeDtypeStruct((N_PAD, D), jnp.float32)),
    mesh=_mesh,
    scratch_types=[
        pltpu.VMEM((CH,), jnp.int32),
        pltpu.VMEM((CH,), jnp.int32),
        pltpu.VMEM((CH, D), jnp.float32),
        pltpu.VMEM_SHARED((N_PAD, D), jnp.float32),
        pltpu.SemaphoreType.DMA,
    ],
)
def _hop_kernel(y_hbm, src_hbm, dst_hbm, zeros2_hbm, p0_hbm, p1_hbm,
                idx_s, idx_d, rows_v, acc, sem):
    c = lax.axis_index("c")
    s = lax.axis_index("s")
    sl = pl.ds(s * ROWS_T, ROWS_T)
    pltpu.sync_copy(zeros2_hbm.at[sl], acc.at[sl])
    plsc.subcore_barrier()

    base = c * E_HALF + s * E_TILE

    def body(i, _):
        pltpu.sync_copy(src_hbm.at[pl.ds(base + i * CH, CH)], idx_s)
        pltpu.sync_copy(dst_hbm.at[pl.ds(base + i * CH, CH)], idx_d)
        pltpu.async_copy(y_hbm.at[idx_s], rows_v, sem).wait()
        pltpu.sync_copy(rows_v, acc.at[idx_d], add=True)
        return 0
    lax.fori_loop(0, NCHUNK, body, 0)
    plsc.subcore_barrier()

    @pl.when(c == 0)
    def _():
        pltpu.sync_copy(acc.at[sl], p0_hbm.at[sl])

    @pl.when(c == 1)
    def _():
        pltpu.sync_copy(acc.at[sl], p1_hbm.at[sl])


RB = 2000


def _scale_body(x_ref, d0_ref, d1_ref, y_ref):
    deg = d0_ref[...] + d1_ref[...] + 1.0
    y_ref[...] = x_ref[...] * lax.rsqrt(deg)


def _mid_body(p0_ref, p1_ref, y_ref, d0_ref, d1_ref, z_ref):
    deg = d0_ref[...] + d1_ref[...] + 1.0
    z_ref[...] = (p0_ref[...] + p1_ref[...] + y_ref[...]) / deg


def _out_body(q0_ref, q1_ref, z_ref, d0_ref, d1_ref, w_ref, b_ref, o_ref):
    deg = d0_ref[...] + d1_ref[...] + 1.0
    h = (q0_ref[...] + q1_ref[...] + z_ref[...]) * lax.rsqrt(deg)
    o_ref[...] = jnp.dot(h, w_ref[...],
                         preferred_element_type=jnp.float32) + b_ref[...]


def _row_spec():
    return pl.BlockSpec((RB, D), lambda i: (i, 0))


def _col_spec():
    return pl.BlockSpec((RB, 1), lambda i: (i, 0))


def kernel(x, edge_index, edge_weights, W, b):
    del edge_weights
    src = edge_index[0]
    dst = edge_index[1]
    pad = E_PAD - E
    src_p = jnp.concatenate([src, jnp.zeros((pad,), jnp.int32)])
    dst_p = jnp.concatenate([dst, jnp.full((pad,), DUMMY, jnp.int32)])
    zeros1 = jnp.zeros((N_PAD,), jnp.float32)
    zeros2 = jnp.zeros((N_PAD, D), jnp.float32)

    d0, d1 = _deg_kernel(dst_p, zeros1)
    d0c = d0.reshape(N_PAD, 1)
    d1c = d1.reshape(N_PAD, 1)

    grid = N // RB
    y = pl.pallas_call(
        _scale_body,
        grid=(grid,),
        in_specs=[_row_spec(), _col_spec(), _col_spec()],
        out_specs=_row_spec(),
        out_shape=jax.ShapeDtypeStruct((N, D), jnp.float32),
    )(x, d0c, d1c)

    p0, p1 = _hop_kernel(y, src_p, dst_p, zeros2)

    z = pl.pallas_call(
        _mid_body,
        grid=(grid,),
        in_specs=[_row_spec(), _row_spec(), _row_spec(),
                  _col_spec(), _col_spec()],
        out_specs=_row_spec(),
        out_shape=jax.ShapeDtypeStruct((N, D), jnp.float32),
    )(p0, p1, y, d0c, d1c)

    q0, q1 = _hop_kernel(z, src_p, dst_p, zeros2)

    out = pl.pallas_call(
        _out_body,
        grid=(grid,),
        in_specs=[_row_spec(), _row_spec(), _row_spec(),
                  _col_spec(), _col_spec(),
                  pl.BlockSpec((D, D), lambda i: (0, 0)),
                  pl.BlockSpec((1, D), lambda i: (0, 0))],
        out_specs=_row_spec(),
        out_shape=jax.ShapeDtypeStruct((N, D), jnp.float32),
    )(q0, q1, z, d0c, d1c, W, b.reshape(1, D))
    return out
